# baseline (speedup 1.0000x reference)
import functools

import jax
import jax.numpy as jnp
from jax import lax
from jax.experimental import pallas as pl
from jax.experimental.pallas import tpu as pltpu
from jax.experimental.pallas import tpu_sc as plsc

LPAD = 52
RR = LPAD // 4


def _sc_gather_hist(item_table, hist_idx_perm, B):
    D = item_table.shape[1]

    info = plsc.get_sparse_core_info()
    NC, NS = info.num_cores, info.num_subcores
    NW = NC * NS

    b_per_w = B // NW
    NB = 32
    CH = NB * LPAD
    n_chunks = b_per_w // NB
    n_sub = CH // 128

    mesh = plsc.VectorSubcoreMesh(core_axis_name="c", subcore_axis_name="s")

    @functools.partial(
        pl.kernel,
        mesh=mesh,
        out_type=jax.ShapeDtypeStruct((RR, B * 4, D), jnp.float32),
        scratch_types=[
            pltpu.VMEM((2 * CH,), jnp.int32),
            pltpu.VMEM((2 * CH, D), jnp.float32),
            pltpu.SemaphoreType.DMA,
            pltpu.SemaphoreType.DMA,
        ],
        compiler_params=pltpu.CompilerParams(use_tc_tiling_on_sc=False),
    )
    def k(item_hbm, hidx_hbm, hout_hbm, idx_v, rows_v, sem, wsem):
        wid = lax.axis_index("s") * NC + lax.axis_index("c")

        def issue_gathers(i, boff):
            c = wid * n_chunks + i
            pltpu.sync_copy(hidx_hbm.at[pl.ds(c * CH, CH)],
                            idx_v.at[pl.ds(boff, CH)])
            for j in range(n_sub):
                pltpu.async_copy(
                    item_hbm.at[idx_v.at[pl.ds(boff + j * 128, 128)]],
                    rows_v.at[pl.ds(boff + j * 128, 128)],
                    sem,
                )

        def drain(semref, boff):
            pltpu.make_async_copy(item_hbm.at[pl.ds(0, CH)],
                                  rows_v.at[pl.ds(boff, CH)],
                                  semref).wait()

        def issue_writes(i, boff):
            b0 = wid * b_per_w + i * NB
            for rr in range(RR):
                pltpu.async_copy(
                    rows_v.at[pl.ds(boff + rr * 4 * NB, 4 * NB)],
                    hout_hbm.at[rr, pl.ds(b0 * 4, 4 * NB)],
                    wsem,
                )

        issue_gathers(0, 0)

        def hist_chunk(i, carry):
            boff = (i % 2) * CH
            alt = CH - boff

            @pl.when(i >= 2)
            def _():
                drain(wsem, boff)

            drain(sem, alt)
            issue_writes(i - 1, alt)
            issue_gathers(i, boff)
            return carry

        lax.fori_loop(1, n_chunks, hist_chunk, 0)

        last = n_chunks - 1
        lboff = (last % 2) * CH
        drain(sem, lboff)
        issue_writes(last, lboff)
        drain(wsem, CH - lboff)
        drain(wsem, lboff)

    return k(item_table, hist_idx_perm)


def _sc_gather_user(user_table, user_idx):
    B = user_idx.shape[0]
    D = user_table.shape[1]

    info = plsc.get_sparse_core_info()
    NC, NS = info.num_cores, info.num_subcores
    NW = NC * NS
    u_per_w = B // NW

    mesh = plsc.VectorSubcoreMesh(core_axis_name="c", subcore_axis_name="s")

    @functools.partial(
        pl.kernel,
        mesh=mesh,
        out_type=jax.ShapeDtypeStruct((B, D), jnp.float32),
        scratch_types=[
            pltpu.VMEM((u_per_w,), jnp.int32),
            pltpu.VMEM((u_per_w, D), jnp.float32),
            pltpu.SemaphoreType.DMA,
        ],
        compiler_params=pltpu.CompilerParams(use_tc_tiling_on_sc=False),
    )
    def k(user_hbm, uidx_hbm, uout_hbm, idx_v, rows_v, sem):
        wid = lax.axis_index("s") * NC + lax.axis_index("c")
        ub = wid * u_per_w
        pltpu.sync_copy(uidx_hbm.at[pl.ds(ub, u_per_w)], idx_v)
        copies = []
        for j in range(u_per_w // 128):
            copies.append(pltpu.async_copy(
                user_hbm.at[idx_v.at[pl.ds(j * 128, 128)]],
                rows_v.at[pl.ds(j * 128, 128)],
                sem,
            ))
        for c in copies:
            c.wait()
        pltpu.sync_copy(rows_v, uout_hbm.at[pl.ds(ub, u_per_w)])

    return k(user_table, user_idx)


def _dense_body(hist_ref, hid_ref, uemb_ref, uid_ref,
                wqt_ref, bq_ref, wk_ref, bk_ref,
                w1ta_ref, w1tb_ref, b1_ref, w2t_ref, b2_ref, out_ref):
    BB = hid_ref.shape[0]
    D = 32
    C = LPAD * D

    m = hid_ref[...] != 0
    mf = m.astype(jnp.float32)

    Xcat = jnp.concatenate([hist_ref[rr] for rr in range(RR)], axis=1)

    ci = lambda sh, dim: lax.broadcasted_iota(jnp.int32, sh, dim)
    E = (ci((LPAD, C), 1) // D == ci((LPAD, C), 0)).astype(jnp.float32)
    ET = (ci((C, LPAD), 0) // D == ci((C, LPAD), 1)).astype(jnp.float32)
    F = (ci((C, D), 0) % D == ci((C, D), 1)).astype(jnp.float32)
    FT = (ci((D, C), 1) % D == ci((D, C), 0)).astype(jnp.float32)

    mm = functools.partial(jnp.dot, preferred_element_type=jnp.float32)

    mf_exp = mm(mf, E)
    sum_hist = mm(Xcat * mf_exp, F)
    count = jnp.maximum(jnp.sum(mf, axis=1, keepdims=True), 1.0)
    query = mm(sum_hist / count, wqt_ref[...]) + bq_ref[...]

    qk = mm(query, wk_ref[...])
    s0 = jnp.sum(query * bk_ref[...], axis=1, keepdims=True)

    scores = mm(Xcat * mm(qk, FT), ET) + s0
    scores = jnp.where(m, scores, -1e9)
    mx = jnp.max(scores, axis=1, keepdims=True)
    e = jnp.exp(scores - mx)
    w = e / jnp.sum(e, axis=1, keepdims=True)

    hist_vec = mm(Xcat * (mm(w, E) * mf_exp), F)

    uemb = uemb_ref[...] * (uid_ref[...] != 0).astype(jnp.float32)
    h = jnp.maximum(
        jnp.dot(uemb, w1ta_ref[...], preferred_element_type=jnp.float32)
        + jnp.dot(hist_vec, w1tb_ref[...], preferred_element_type=jnp.float32)
        + b1_ref[...], 0.0)
    out = (jnp.dot(h, w2t_ref[...], preferred_element_type=jnp.float32)
           + b2_ref[...])
    norm = jnp.sqrt(jnp.sum(out * out, axis=1, keepdims=True))
    out_ref[...] = out / jnp.maximum(norm, 1e-12)


def _tc_dense(hist_slabs, hid52, user_emb, user_id,
              WqT, bq, Wk, bk, W1Ta, W1Tb, b1, W2T, b2, block_b=256):
    B = hid52.shape[0]
    D = 32
    H = W2T.shape[0]
    grid = (B // block_b,)

    full = lambda shape: pl.BlockSpec(shape, lambda i: (0,) * len(shape))
    return pl.pallas_call(
        _dense_body,
        grid=grid,
        in_specs=[
            pl.BlockSpec((RR, block_b, 128), lambda i: (0, i, 0)),
            pl.BlockSpec((block_b, LPAD), lambda i: (i, 0)),
            pl.BlockSpec((block_b, D), lambda i: (i, 0)),
            pl.BlockSpec((block_b, 1), lambda i: (i, 0)),
            full((D, D)), full((1, D)),
            full((D, D)), full((1, D)),
            full((D, H)), full((D, H)), full((1, H)),
            full((H, D)), full((1, D)),
        ],
        out_specs=pl.BlockSpec((block_b, D), lambda i: (i, 0)),
        out_shape=jax.ShapeDtypeStruct((B, D), jnp.float32),
    )(hist_slabs, hid52, user_emb, user_id,
      WqT, bq, Wk, bk, W1Ta, W1Tb, b1, W2T, b2)


def kernel(user_table, item_table, Wq, bq, Wk, bk, W1, b1, W2, b2,
           user_id, hist_article_id):
    B, L = hist_article_id.shape
    D = user_table.shape[1]
    H = W1.shape[0]

    uid = user_id.astype(jnp.int32)
    hid = hist_article_id.astype(jnp.int32)
    hid52 = jnp.pad(hid, ((0, 0), (0, LPAD - L)))

    NB = 32
    hid_perm = (hid52.reshape(B // NB, NB, RR, 4)
                .transpose(0, 2, 1, 3).reshape(B * LPAD))

    user_emb = _sc_gather_user(user_table, uid)
    hist4 = _sc_gather_hist(item_table, hid_perm, B)
    hist_slabs = hist4.reshape(RR, B, 128)

    W1T = W1.T
    return _tc_dense(
        hist_slabs, hid52, user_emb, uid.reshape(B, 1),
        Wq.T, bq.reshape(1, D), Wk, bk.reshape(1, D),
        W1T[:D], W1T[D:], b1.reshape(1, H), W2.T, b2.reshape(1, D),
    )


# speedup vs baseline: 1.0661x; 1.0661x over previous
import functools

import jax
import jax.numpy as jnp
from jax import lax
from jax.experimental import pallas as pl
from jax.experimental.pallas import tpu as pltpu
from jax.experimental.pallas import tpu_sc as plsc

LPAD = 52
RR = LPAD // 4


def _sc_gather(item_table, user_table, hist_idx_perm, user_idx):
    B = user_idx.shape[0]
    D = item_table.shape[1]

    info = plsc.get_sparse_core_info()
    NC, NS = info.num_cores, info.num_subcores
    NW = NC * NS

    b_per_w = B // NW
    NB = 32
    CH = NB * LPAD
    n_chunks = b_per_w // NB
    n_sub = CH // 128

    u_per_w = B // NW

    mesh = plsc.VectorSubcoreMesh(core_axis_name="c", subcore_axis_name="s")

    @functools.partial(
        pl.kernel,
        mesh=mesh,
        out_type=[
            jax.ShapeDtypeStruct((RR, B * 4, D), jnp.float32),
            jax.ShapeDtypeStruct((B, D), jnp.float32),
        ],
        scratch_types=[
            pltpu.VMEM((CH,), jnp.int32),
            pltpu.VMEM((CH, D), jnp.float32),
            pltpu.SemaphoreType.DMA,
            pltpu.SemaphoreType.DMA,
        ],
        compiler_params=pltpu.CompilerParams(use_tc_tiling_on_sc=False),
    )
    def k(item_hbm, user_hbm, hidx_hbm, uidx_hbm, hout_hbm, uout_hbm,
          idx_v, rows_v, sem, wsem):
        wid = lax.axis_index("s") * NC + lax.axis_index("c")

        def hist_chunk(i, carry):
            c = wid * n_chunks + i
            b0 = wid * b_per_w + i * NB
            pltpu.sync_copy(hidx_hbm.at[pl.ds(c * CH, CH)], idx_v)
            copies = []
            for j in range(n_sub):
                copies.append(pltpu.async_copy(
                    item_hbm.at[idx_v.at[pl.ds(j * 128, 128)]],
                    rows_v.at[pl.ds(j * 128, 128)],
                    sem,
                ))
            for cp in copies:
                cp.wait()
            wcopies = []
            for rr in range(RR):
                wcopies.append(pltpu.async_copy(
                    rows_v.at[pl.ds(rr * 4 * NB, 4 * NB)],
                    hout_hbm.at[rr, pl.ds(b0 * 4, 4 * NB)],
                    wsem,
                ))
            for cp in wcopies:
                cp.wait()
            return carry

        lax.fori_loop(0, n_chunks, hist_chunk, 0)

        ub = wid * u_per_w
        pltpu.sync_copy(uidx_hbm.at[pl.ds(ub, u_per_w)], idx_v.at[pl.ds(0, u_per_w)])
        copies = []
        for j in range(u_per_w // 128):
            copies.append(pltpu.async_copy(
                user_hbm.at[idx_v.at[pl.ds(j * 128, 128)]],
                rows_v.at[pl.ds(j * 128, 128)],
                sem,
            ))
        for c in copies:
            c.wait()
        pltpu.sync_copy(rows_v.at[pl.ds(0, u_per_w)],
                        uout_hbm.at[pl.ds(ub, u_per_w)])

    return k(item_table, user_table, hist_idx_perm, user_idx)


def _dense_body(hist_ref, hid_ref, uemb_ref, uid_ref,
                wqt_ref, bq_ref, wk_ref, bk_ref,
                w1ta_ref, w1tb_ref, b1_ref, w2t_ref, b2_ref, out_ref):
    BB = hid_ref.shape[0]
    D = 32
    C = LPAD * D

    m = hid_ref[...] != 0
    mf = m.astype(jnp.float32)

    Xcat = jnp.concatenate([hist_ref[rr] for rr in range(RR)], axis=1)

    ci = lambda sh, dim: lax.broadcasted_iota(jnp.int32, sh, dim)
    E = (ci((LPAD, C), 1) // D == ci((LPAD, C), 0)).astype(jnp.float32)
    ET = (ci((C, LPAD), 0) // D == ci((C, LPAD), 1)).astype(jnp.float32)
    F = (ci((C, D), 0) % D == ci((C, D), 1)).astype(jnp.float32)
    FT = (ci((D, C), 1) % D == ci((D, C), 0)).astype(jnp.float32)

    mm = functools.partial(jnp.dot, preferred_element_type=jnp.float32)

    mf_exp = mm(mf, E)
    sum_hist = mm(Xcat * mf_exp, F)
    count = jnp.maximum(jnp.sum(mf, axis=1, keepdims=True), 1.0)
    query = mm(sum_hist / count, wqt_ref[...]) + bq_ref[...]

    qk = mm(query, wk_ref[...])
    s0 = jnp.sum(query * bk_ref[...], axis=1, keepdims=True)

    scores = mm(Xcat * mm(qk, FT), ET) + s0
    scores = jnp.where(m, scores, -1e9)
    mx = jnp.max(scores, axis=1, keepdims=True)
    e = jnp.exp(scores - mx)
    w = e / jnp.sum(e, axis=1, keepdims=True)

    hist_vec = mm(Xcat * (mm(w, E) * mf_exp), F)

    uemb = uemb_ref[...] * (uid_ref[...] != 0).astype(jnp.float32)
    h = jnp.maximum(
        jnp.dot(uemb, w1ta_ref[...], preferred_element_type=jnp.float32)
        + jnp.dot(hist_vec, w1tb_ref[...], preferred_element_type=jnp.float32)
        + b1_ref[...], 0.0)
    out = (jnp.dot(h, w2t_ref[...], preferred_element_type=jnp.float32)
           + b2_ref[...])
    norm = jnp.sqrt(jnp.sum(out * out, axis=1, keepdims=True))
    out_ref[...] = out / jnp.maximum(norm, 1e-12)


def _tc_dense(hist_slabs, hid52, user_emb, user_id,
              WqT, bq, Wk, bk, W1Ta, W1Tb, b1, W2T, b2, block_b=256):
    B = hid52.shape[0]
    D = 32
    H = W2T.shape[0]
    grid = (B // block_b,)

    full = lambda shape: pl.BlockSpec(shape, lambda i: (0,) * len(shape))
    return pl.pallas_call(
        _dense_body,
        grid=grid,
        in_specs=[
            pl.BlockSpec((RR, block_b, 128), lambda i: (0, i, 0)),
            pl.BlockSpec((block_b, LPAD), lambda i: (i, 0)),
            pl.BlockSpec((block_b, D), lambda i: (i, 0)),
            pl.BlockSpec((block_b, 1), lambda i: (i, 0)),
            full((D, D)), full((1, D)),
            full((D, D)), full((1, D)),
            full((D, H)), full((D, H)), full((1, H)),
            full((H, D)), full((1, D)),
        ],
        out_specs=pl.BlockSpec((block_b, D), lambda i: (i, 0)),
        out_shape=jax.ShapeDtypeStruct((B, D), jnp.float32),
    )(hist_slabs, hid52, user_emb, user_id,
      WqT, bq, Wk, bk, W1Ta, W1Tb, b1, W2T, b2)


def kernel(user_table, item_table, Wq, bq, Wk, bk, W1, b1, W2, b2,
           user_id, hist_article_id):
    B, L = hist_article_id.shape
    D = user_table.shape[1]
    H = W1.shape[0]

    uid = user_id.astype(jnp.int32)
    hid = hist_article_id.astype(jnp.int32)
    hid52 = jnp.pad(hid, ((0, 0), (0, LPAD - L)))

    NB = 32
    hid_perm = (hid52.reshape(B // NB, NB, RR, 4)
                .transpose(0, 2, 1, 3).reshape(B * LPAD))

    hist4, user_emb = _sc_gather(item_table, user_table, hid_perm, uid)
    hist_slabs = hist4.reshape(RR, B, 128)

    W1T = W1.T
    return _tc_dense(
        hist_slabs, hid52, user_emb, uid.reshape(B, 1),
        Wq.T, bq.reshape(1, D), Wk, bk.reshape(1, D),
        W1T[:D], W1T[D:], b1.reshape(1, H), W2.T, b2.reshape(1, D),
    )
